# baseline (speedup 1.0000x reference)
import jax
import jax.numpy as jnp
from jax import lax
from jax.experimental import pallas as pl
from jax.experimental.pallas import tpu as pltpu
from jax.experimental.pallas import tpu_sc as plsc

DIM = 64
BATCH = 16384

_NC, _NS, _L = 2, 16, 16
_NW = _NC * _NS
_BPW = BATCH // _NW
_CHUNK = 128
_NCHUNK = _BPW // _CHUNK


def _sc_kernel(uid_hbm, iid_hbm, p_hbm, q_hbm, out_hbm,
               uid_v, iid_v, p_v, q_v, out_v, sem):
    wid = lax.axis_index("s") * _NC + lax.axis_index("c")
    pltpu.sync_copy(uid_hbm.at[pl.ds(wid * _NCHUNK, _NCHUNK)], uid_v)
    pltpu.sync_copy(iid_hbm.at[pl.ds(wid * _NCHUNK, _NCHUNK)], iid_v)

    copies = []
    for c in range(_NCHUNK):
        copies.append(pltpu.async_copy(
            p_hbm.at[uid_v.at[c]], p_v.at[pl.ds(c * _CHUNK, _CHUNK)], sem))
        copies.append(pltpu.async_copy(
            q_hbm.at[iid_v.at[c]], q_v.at[pl.ds(c * _CHUNK, _CHUNK)], sem))
    for cp in copies:
        cp.wait()

    lanes = lax.iota(jnp.int32, _L)
    idx_e = (lanes * 2) % _L
    idx_o = idx_e + 1
    half = lanes < (_L // 2)

    def hadd(a, b):
        ae = a.at[idx_e].get(mode="promise_in_bounds")
        ao = a.at[idx_o].get(mode="promise_in_bounds")
        be = b.at[idx_e].get(mode="promise_in_bounds")
        bo = b.at[idx_o].get(mode="promise_in_bounds")
        return jnp.where(half, ae + ao, be + bo)

    def body(g, carry):
        accs = []
        for r in range(_L):
            acc = None
            for c in range(DIM // _L):
                pv = p_v[g * _L + r, pl.ds(c * _L, _L)]
                qv = q_v[g * _L + r, pl.ds(c * _L, _L)]
                acc = pv * qv if acc is None else acc + pv * qv
            accs.append(acc)
        while len(accs) > 1:
            accs = [hadd(accs[i], accs[i + 1]) for i in range(0, len(accs), 2)]
        acc = jnp.minimum(jnp.maximum(accs[0], 0.0), 1.0)
        out_v[pl.ds(g * _L, _L)] = acc
        return carry

    lax.fori_loop(0, _BPW // _L, body, 0)
    pltpu.sync_copy(out_v, out_hbm.at[pl.ds(wid * _BPW, _BPW)])


@jax.jit
def kernel(user_id, item_id, P, Q):
    mesh = plsc.VectorSubcoreMesh(core_axis_name="c", subcore_axis_name="s")
    run = pl.kernel(
        _sc_kernel,
        out_type=jax.ShapeDtypeStruct((BATCH,), jnp.float32),
        mesh=mesh,
        scratch_types=[
            pltpu.VMEM((_NCHUNK, _CHUNK), jnp.int32),
            pltpu.VMEM((_NCHUNK, _CHUNK), jnp.int32),
            pltpu.VMEM((_BPW, DIM), jnp.float32),
            pltpu.VMEM((_BPW, DIM), jnp.float32),
            pltpu.VMEM((_BPW,), jnp.float32),
            pltpu.SemaphoreType.DMA,
        ],
        compiler_params=pltpu.CompilerParams(use_tc_tiling_on_sc=False),
    )
    uid2d = user_id.astype(jnp.int32).reshape(BATCH // _CHUNK, _CHUNK)
    iid2d = item_id.astype(jnp.int32).reshape(BATCH // _CHUNK, _CHUNK)
    score = run(uid2d, iid2d, P, Q)
    return score[:, None]


# speedup vs baseline: 2.7414x; 2.7414x over previous
import jax
import jax.numpy as jnp
from jax import lax
from jax.experimental import pallas as pl
from jax.experimental.pallas import tpu as pltpu
from jax.experimental.pallas import tpu_sc as plsc

DIM = 64
BATCH = 16384
NU = 1000000

_NC, _NS, _L = 2, 16, 16
_NW = _NC * _NS
_BPW = BATCH // _NW
_NB = (NU + 127) // 128
_MAXCL = (_NB + _NW - 1) // _NW
_CPAD = 256
_RING = 8


def _gather_kernel(uid_hbm, iid_hbm, pt_hbm, qt_hbm, gp_hbm, gq_hbm,
                   idx_v, us_v, bs_v, cnt_v, off_v, cur_v,
                   s0, s1, stg,
                   semA, semB, semO):
    wid = lax.axis_index("s") * _NC + lax.axis_index("c")
    lanes = lax.iota(jnp.int32, _L)
    ones = jnp.ones((_L,), jnp.int32)
    zeros = jnp.zeros((_L,), jnp.int32)

    n_cl = (_NB - 1 - wid) // _NW + 1

    drainA = pltpu.make_async_copy(
        pt_hbm.at[:, pl.ds(0, 128)], s0, semA)
    drainB = pltpu.make_async_copy(
        pt_hbm.at[:, pl.ds(0, 128)], s1, semB)
    drainO = pltpu.make_async_copy(
        gp_hbm.at[pl.ds(0, DIM)], stg.at[pl.ds(0, DIM)], semO)

    for src_hbm, tab_hbm, out_hbm in ((uid_hbm, pt_hbm, gp_hbm),
                                      (iid_hbm, qt_hbm, gq_hbm)):
        pltpu.sync_copy(src_hbm, idx_v)

        def zero_cnt(j, carry):
            cnt_v[pl.ds(j * _L, _L)] = zeros
            return carry

        lax.fori_loop(0, _CPAD // _L, zero_cnt, 0)

        def count(i, carry):
            u = idx_v[pl.ds(i * _L, _L)]
            c = u >> 7
            mine = (c & (_NW - 1)) == wid
            cl = c >> 5
            plsc.addupdate_scatter(cnt_v, [cl], ones, mask=mine)
            return carry

        lax.fori_loop(0, BATCH // _L, count, 0)

        def pfx(j, carry):
            chunk = cnt_v[pl.ds(j * _L, _L)]
            cs = plsc.cumsum(chunk)
            off_v[pl.ds(j * _L, _L)] = cs - chunk + carry
            return carry + cs[_L - 1]

        lax.fori_loop(0, _CPAD // _L, pfx, 0)

        def pcopy(j, carry):
            cur_v[pl.ds(j * _L, _L)] = off_v[pl.ds(j * _L, _L)]
            return carry

        lax.fori_loop(0, _CPAD // _L, pcopy, 0)

        def place(i, carry):
            u = idx_v[pl.ds(i * _L, _L)]
            c = u >> 7
            mine = (c & (_NW - 1)) == wid
            cl = c >> 5
            base = plsc.load_gather(cur_v, [cl])
            rank, last = plsc.scan_count(cl, mask=mine)
            pos = base + rank - 1
            plsc.store_scatter(us_v, [pos], u, mask=mine)
            plsc.store_scatter(bs_v, [pos], i * _L + lanes, mask=mine)
            lastm = jnp.logical_and(mine, last)
            plsc.store_scatter(cur_v, [cl], base + rank, mask=lastm)
            return carry

        lax.fori_loop(0, BATCH // _L, place, 0)

        def fetch(cl, buf, sem):
            c = cl * _NW + wid
            pltpu.async_copy(tab_hbm.at[:, pl.ds(c * 128, 128)], buf, sem)

        fetch(0, s0, semA)

        def vsplat(ref, e):
            vec = ref[pl.ds((e // _L) * _L, _L)]
            sel = jnp.full((_L,), 0, jnp.int32) + (e % _L)
            return vec.at[sel].get(mode="promise_in_bounds")

        def scal(ref, e):
            return vsplat(ref, e)[0]

        def extract(cl, buf, j0):
            n = scal(cnt_v, cl)
            o = scal(off_v, cl)
            cbase = (cl * _NW + wid) * 128

            def one(e, j):
                u = vsplat(us_v, e)
                lane = u - cbase
                b0 = scal(bs_v, e)
                jmod = j % _RING
                for d16 in range(DIM // _L):
                    v = plsc.load_gather(buf, [d16 * _L + lanes, lane])
                    stg[pl.ds(jmod * DIM + d16 * _L, _L)] = v

                @pl.when(j >= _RING)
                def _():
                    drainO.wait()

                pltpu.async_copy(stg.at[pl.ds(jmod * DIM, DIM)],
                                 out_hbm.at[pl.ds(b0 * DIM, DIM)], semO)
                return j + 1

            return lax.fori_loop(o, o + n, one, j0)

        def step(cl, j):
            @pl.when(cl + 1 < n_cl)
            def _():
                @pl.when((cl + 1) % 2 == 0)
                def _():
                    fetch(cl + 1, s0, semA)

                @pl.when((cl + 1) % 2 == 1)
                def _():
                    fetch(cl + 1, s1, semB)

            def even(j):
                drainA.wait()
                return extract(cl, s0, j)

            def odd(j):
                drainB.wait()
                return extract(cl, s1, j)

            return lax.cond(cl % 2 == 0, even, odd, j)

        j_end = lax.fori_loop(0, n_cl, step, 0)

        rem = jnp.minimum(j_end, _RING)
        lax.fori_loop(0, rem, lambda i, c: (drainO.wait(), c)[1], 0)


def _dot_kernel(gp_hbm, gq_hbm, out_hbm, p_v, q_v, out_v, sem):
    wid = lax.axis_index("s") * _NC + lax.axis_index("c")
    base = wid * _BPW
    pltpu.sync_copy(gp_hbm.at[pl.ds(base * DIM, _BPW * DIM)], p_v)
    pltpu.sync_copy(gq_hbm.at[pl.ds(base * DIM, _BPW * DIM)], q_v)

    lanes = lax.iota(jnp.int32, _L)
    idx_e = (lanes * 2) % _L
    idx_o = idx_e + 1
    half_m = lanes < (_L // 2)

    def hadd(a, b):
        ae = a.at[idx_e].get(mode="promise_in_bounds")
        ao = a.at[idx_o].get(mode="promise_in_bounds")
        be = b.at[idx_e].get(mode="promise_in_bounds")
        bo = b.at[idx_o].get(mode="promise_in_bounds")
        return jnp.where(half_m, ae + ao, be + bo)

    def body(g, carry):
        accs = []
        for k in range(_L):
            acc = None
            for c in range(DIM // _L):
                o = (g * _L + k) * DIM + c * _L
                pv = p_v[pl.ds(o, _L)]
                qv = q_v[pl.ds(o, _L)]
                acc = pv * qv if acc is None else acc + pv * qv
            accs.append(acc)
        while len(accs) > 1:
            accs = [hadd(accs[i], accs[i + 1]) for i in range(0, len(accs), 2)]
        acc = jnp.minimum(jnp.maximum(accs[0], 0.0), 1.0)
        out_v[pl.ds(g * _L, _L)] = acc
        return carry

    lax.fori_loop(0, _BPW // _L, body, 0)
    pltpu.sync_copy(out_v, out_hbm.at[pl.ds(base, _BPW)])


@jax.jit
def kernel(user_id, item_id, P, Q):
    mesh = plsc.VectorSubcoreMesh(core_axis_name="c", subcore_axis_name="s")
    gather = pl.kernel(
        _gather_kernel,
        out_type=(jax.ShapeDtypeStruct((BATCH * DIM,), jnp.float32),
                  jax.ShapeDtypeStruct((BATCH * DIM,), jnp.float32)),
        mesh=mesh,
        scratch_types=[
            pltpu.VMEM((BATCH,), jnp.int32),
            pltpu.VMEM((BATCH,), jnp.int32),
            pltpu.VMEM((BATCH,), jnp.int32),
            pltpu.VMEM((_CPAD,), jnp.int32),
            pltpu.VMEM((_CPAD,), jnp.int32),
            pltpu.VMEM((_CPAD,), jnp.int32),
            pltpu.VMEM((DIM, 128), jnp.float32),
            pltpu.VMEM((DIM, 128), jnp.float32),
            pltpu.VMEM((_RING * DIM,), jnp.float32),
            pltpu.SemaphoreType.DMA,
            pltpu.SemaphoreType.DMA,
            pltpu.SemaphoreType.DMA,
        ],
        compiler_params=pltpu.CompilerParams(needs_layout_passes=False),
    )
    dot = pl.kernel(
        _dot_kernel,
        out_type=jax.ShapeDtypeStruct((BATCH,), jnp.float32),
        mesh=mesh,
        scratch_types=[
            pltpu.VMEM((_BPW * DIM,), jnp.float32),
            pltpu.VMEM((_BPW * DIM,), jnp.float32),
            pltpu.VMEM((_BPW,), jnp.float32),
            pltpu.SemaphoreType.DMA,
        ],
    )
    gp, gq = gather(user_id.astype(jnp.int32), item_id.astype(jnp.int32),
                    P.T, Q.T)
    score = dot(gp, gq)
    return score[:, None]


# speedup vs baseline: 3.7070x; 1.3522x over previous
import jax
import jax.numpy as jnp
from jax import lax
from jax.experimental import pallas as pl
from jax.experimental.pallas import tpu as pltpu
from jax.experimental.pallas import tpu_sc as plsc

DIM = 64
BATCH = 16384
NU = 1000000

_NC, _NS, _L = 2, 16, 16
_NW = _NC * _NS
_BPW = BATCH // _NW
_NB = (NU + 127) // 128
_MAXCL = (_NB + _NW - 1) // _NW
_CPAD = 256
_RING = 8


def _gather_kernel(uid_hbm, iid_hbm, pt_hbm, qt_hbm, gp_hbm, gq_hbm,
                   idx_v, us_v, bs_v, cnt_v, off_v, cur_v,
                   s0, s1, s2, s3, stg,
                   semA, semB, semC, semD, semO):
    wid = lax.axis_index("s") * _NC + lax.axis_index("c")
    lanes = lax.iota(jnp.int32, _L)
    ones = jnp.ones((_L,), jnp.int32)
    zeros = jnp.zeros((_L,), jnp.int32)

    n_cl = (_NB - 1 - wid) // _NW + 1

    bufs = (s0, s1, s2, s3)
    sems = (semA, semB, semC, semD)
    drains = tuple(
        pltpu.make_async_copy(pt_hbm.at[:, pl.ds(0, 128)], b, s)
        for b, s in zip(bufs, sems))
    drainO = pltpu.make_async_copy(
        gp_hbm.at[pl.ds(0, DIM)], stg.at[pl.ds(0, DIM)], semO)

    for src_hbm, tab_hbm, out_hbm in ((uid_hbm, pt_hbm, gp_hbm),
                                      (iid_hbm, qt_hbm, gq_hbm)):
        pltpu.sync_copy(src_hbm, idx_v)

        def zero_cnt(j, carry):
            cnt_v[pl.ds(j * _L, _L)] = zeros
            return carry

        lax.fori_loop(0, _CPAD // _L, zero_cnt, 0)

        def count(i, carry):
            u = idx_v[pl.ds(i * _L, _L)]
            c = u >> 7
            mine = (c & (_NW - 1)) == wid
            cl = c >> 5
            plsc.addupdate_scatter(cnt_v, [cl], ones, mask=mine)
            return carry

        lax.fori_loop(0, BATCH // _L, count, 0)

        def pfx(j, carry):
            chunk = cnt_v[pl.ds(j * _L, _L)]
            cs = plsc.cumsum(chunk)
            off_v[pl.ds(j * _L, _L)] = cs - chunk + carry
            return carry + cs[_L - 1]

        lax.fori_loop(0, _CPAD // _L, pfx, 0)

        def pcopy(j, carry):
            cur_v[pl.ds(j * _L, _L)] = off_v[pl.ds(j * _L, _L)]
            return carry

        lax.fori_loop(0, _CPAD // _L, pcopy, 0)

        def place(i, carry):
            u = idx_v[pl.ds(i * _L, _L)]
            c = u >> 7
            mine = (c & (_NW - 1)) == wid
            cl = c >> 5
            base = plsc.load_gather(cur_v, [cl])
            rank, last = plsc.scan_count(cl, mask=mine)
            pos = base + rank - 1
            plsc.store_scatter(us_v, [pos], u, mask=mine)
            plsc.store_scatter(bs_v, [pos], i * _L + lanes, mask=mine)
            lastm = jnp.logical_and(mine, last)
            plsc.store_scatter(cur_v, [cl], base + rank, mask=lastm)
            return carry

        lax.fori_loop(0, BATCH // _L, place, 0)

        NBUF = len(bufs)

        def fetch(cl, buf, sem):
            c = cl * _NW + wid
            pltpu.async_copy(tab_hbm.at[:, pl.ds(c * 128, 128)], buf, sem)

        def fetch_mod(cl):
            for m in range(NBUF):
                @pl.when(cl % NBUF == m)
                def _(m=m):
                    fetch(cl, bufs[m], sems[m])

        for w in range(NBUF - 1):
            @pl.when(w < n_cl)
            def _(w=w):
                fetch(w, bufs[w], sems[w])

        def vsplat(ref, e):
            vec = ref[pl.ds((e // _L) * _L, _L)]
            sel = jnp.full((_L,), 0, jnp.int32) + (e % _L)
            return vec.at[sel].get(mode="promise_in_bounds")

        def scal(ref, e):
            return vsplat(ref, e)[0]

        def extract(cl, buf, j0):
            n = scal(cnt_v, cl)
            o = scal(off_v, cl)
            cbase = (cl * _NW + wid) * 128

            def one(e, j):
                u = vsplat(us_v, e)
                lane = u - cbase
                b0 = scal(bs_v, e)
                jmod = j % _RING
                for d16 in range(DIM // _L):
                    v = plsc.load_gather(buf, [d16 * _L + lanes, lane])
                    stg[pl.ds(jmod * DIM + d16 * _L, _L)] = v

                @pl.when(j >= _RING)
                def _():
                    drainO.wait()

                pltpu.async_copy(stg.at[pl.ds(jmod * DIM, DIM)],
                                 out_hbm.at[pl.ds(b0 * DIM, DIM)], semO)
                return j + 1

            return lax.fori_loop(o, o + n, one, j0)

        def step(cl, j):
            @pl.when(cl + (NBUF - 1) < n_cl)
            def _():
                fetch_mod(cl + (NBUF - 1))

            def mk(m):
                def br(j):
                    drains[m].wait()
                    return extract(cl, bufs[m], j)
                return br

            return lax.switch(cl % NBUF, [mk(m) for m in range(NBUF)], j)

        j_end = lax.fori_loop(0, n_cl, step, 0)

        rem = jnp.minimum(j_end, _RING)
        lax.fori_loop(0, rem, lambda i, c: (drainO.wait(), c)[1], 0)


def _dot_kernel(gp_hbm, gq_hbm, out_hbm, p_v, q_v, out_v, sem):
    wid = lax.axis_index("s") * _NC + lax.axis_index("c")
    base = wid * _BPW
    pltpu.sync_copy(gp_hbm.at[pl.ds(base * DIM, _BPW * DIM)], p_v)
    pltpu.sync_copy(gq_hbm.at[pl.ds(base * DIM, _BPW * DIM)], q_v)

    lanes = lax.iota(jnp.int32, _L)
    idx_e = (lanes * 2) % _L
    idx_o = idx_e + 1
    half_m = lanes < (_L // 2)

    def hadd(a, b):
        ae = a.at[idx_e].get(mode="promise_in_bounds")
        ao = a.at[idx_o].get(mode="promise_in_bounds")
        be = b.at[idx_e].get(mode="promise_in_bounds")
        bo = b.at[idx_o].get(mode="promise_in_bounds")
        return jnp.where(half_m, ae + ao, be + bo)

    def body(g, carry):
        accs = []
        for k in range(_L):
            acc = None
            for c in range(DIM // _L):
                o = (g * _L + k) * DIM + c * _L
                pv = p_v[pl.ds(o, _L)]
                qv = q_v[pl.ds(o, _L)]
                acc = pv * qv if acc is None else acc + pv * qv
            accs.append(acc)
        while len(accs) > 1:
            accs = [hadd(accs[i], accs[i + 1]) for i in range(0, len(accs), 2)]
        acc = jnp.minimum(jnp.maximum(accs[0], 0.0), 1.0)
        out_v[pl.ds(g * _L, _L)] = acc
        return carry

    lax.fori_loop(0, _BPW // _L, body, 0)
    pltpu.sync_copy(out_v, out_hbm.at[pl.ds(base, _BPW)])


@jax.jit
def kernel(user_id, item_id, P, Q):
    mesh = plsc.VectorSubcoreMesh(core_axis_name="c", subcore_axis_name="s")
    gather = pl.kernel(
        _gather_kernel,
        out_type=(jax.ShapeDtypeStruct((BATCH * DIM,), jnp.float32),
                  jax.ShapeDtypeStruct((BATCH * DIM,), jnp.float32)),
        mesh=mesh,
        scratch_types=[
            pltpu.VMEM((BATCH,), jnp.int32),
            pltpu.VMEM((BATCH,), jnp.int32),
            pltpu.VMEM((BATCH,), jnp.int32),
            pltpu.VMEM((_CPAD,), jnp.int32),
            pltpu.VMEM((_CPAD,), jnp.int32),
            pltpu.VMEM((_CPAD,), jnp.int32),
            pltpu.VMEM((DIM, 128), jnp.float32),
            pltpu.VMEM((DIM, 128), jnp.float32),
            pltpu.VMEM((DIM, 128), jnp.float32),
            pltpu.VMEM((DIM, 128), jnp.float32),
            pltpu.VMEM((_RING * DIM,), jnp.float32),
            pltpu.SemaphoreType.DMA,
            pltpu.SemaphoreType.DMA,
            pltpu.SemaphoreType.DMA,
            pltpu.SemaphoreType.DMA,
            pltpu.SemaphoreType.DMA,
        ],
        compiler_params=pltpu.CompilerParams(needs_layout_passes=False),
    )
    dot = pl.kernel(
        _dot_kernel,
        out_type=jax.ShapeDtypeStruct((BATCH,), jnp.float32),
        mesh=mesh,
        scratch_types=[
            pltpu.VMEM((_BPW * DIM,), jnp.float32),
            pltpu.VMEM((_BPW * DIM,), jnp.float32),
            pltpu.VMEM((_BPW,), jnp.float32),
            pltpu.SemaphoreType.DMA,
        ],
    )
    gp, gq = gather(user_id.astype(jnp.int32), item_id.astype(jnp.int32),
                    P.T, Q.T)
    score = dot(gp, gq)
    return score[:, None]


# speedup vs baseline: 4.1141x; 1.1098x over previous
import jax
import jax.numpy as jnp
from jax import lax
from jax.experimental import pallas as pl
from jax.experimental.pallas import tpu as pltpu
from jax.experimental.pallas import tpu_sc as plsc

DIM = 64
BATCH = 16384
NU = 1000000

_NC, _NS, _L = 2, 16, 16
_NW = _NC * _NS
_BPW = BATCH // _NW
_NB = (NU + 127) // 128
_MAXCL = (_NB + _NW - 1) // _NW
_CPAD = 256
_RING = 8


def _gather_kernel(uid_hbm, iid_hbm, pt_hbm, qt_hbm, gp_hbm, gq_hbm,
                   idx_v, us_v, bs_v, cnt_v, off_v, cur_v,
                   s0, s1, s2, s3, s4, s5, s6, s7, stg,
                   semA, semB, semC, semD, semE, semF, semG, semH, semO):
    wid = lax.axis_index("s") * _NC + lax.axis_index("c")
    lanes = lax.iota(jnp.int32, _L)
    ones = jnp.ones((_L,), jnp.int32)
    zeros = jnp.zeros((_L,), jnp.int32)

    n_cl = (_NB - 1 - wid) // _NW + 1

    bufs = (s0, s1, s2, s3, s4, s5, s6, s7)
    sems = (semA, semB, semC, semD, semE, semF, semG, semH)
    drains = tuple(
        pltpu.make_async_copy(pt_hbm.at[:, pl.ds(0, 128)], b, s)
        for b, s in zip(bufs, sems))
    drainO = pltpu.make_async_copy(
        gp_hbm.at[pl.ds(0, DIM)], stg.at[pl.ds(0, DIM)], semO)

    for src_hbm, tab_hbm, out_hbm in ((uid_hbm, pt_hbm, gp_hbm),
                                      (iid_hbm, qt_hbm, gq_hbm)):
        pltpu.sync_copy(src_hbm, idx_v)

        def zero_cnt(j, carry):
            cnt_v[pl.ds(j * _L, _L)] = zeros
            return carry

        lax.fori_loop(0, _CPAD // _L, zero_cnt, 0)

        def count(i, carry):
            u = idx_v[pl.ds(i * _L, _L)]
            c = u >> 7
            mine = (c & (_NW - 1)) == wid
            cl = c >> 5
            plsc.addupdate_scatter(cnt_v, [cl], ones, mask=mine)
            return carry

        lax.fori_loop(0, BATCH // _L, count, 0)

        def pfx(j, carry):
            chunk = cnt_v[pl.ds(j * _L, _L)]
            cs = plsc.cumsum(chunk)
            off_v[pl.ds(j * _L, _L)] = cs - chunk + carry
            return carry + cs[_L - 1]

        lax.fori_loop(0, _CPAD // _L, pfx, 0)

        def pcopy(j, carry):
            cur_v[pl.ds(j * _L, _L)] = off_v[pl.ds(j * _L, _L)]
            return carry

        lax.fori_loop(0, _CPAD // _L, pcopy, 0)

        def place(i, carry):
            u = idx_v[pl.ds(i * _L, _L)]
            c = u >> 7
            mine = (c & (_NW - 1)) == wid
            cl = c >> 5
            base = plsc.load_gather(cur_v, [cl])
            rank, last = plsc.scan_count(cl, mask=mine)
            pos = base + rank - 1
            plsc.store_scatter(us_v, [pos], u, mask=mine)
            plsc.store_scatter(bs_v, [pos], i * _L + lanes, mask=mine)
            lastm = jnp.logical_and(mine, last)
            plsc.store_scatter(cur_v, [cl], base + rank, mask=lastm)
            return carry

        lax.fori_loop(0, BATCH // _L, place, 0)

        NBUF = len(bufs)

        def fetch(cl, buf, sem):
            c = cl * _NW + wid
            pltpu.async_copy(tab_hbm.at[:, pl.ds(c * 128, 128)], buf, sem)

        def fetch_mod(cl):
            for m in range(NBUF):
                @pl.when(cl % NBUF == m)
                def _(m=m):
                    fetch(cl, bufs[m], sems[m])

        for w in range(NBUF - 1):
            @pl.when(w < n_cl)
            def _(w=w):
                fetch(w, bufs[w], sems[w])

        def vsplat(ref, e):
            vec = ref[pl.ds((e // _L) * _L, _L)]
            sel = jnp.full((_L,), 0, jnp.int32) + (e % _L)
            return vec.at[sel].get(mode="promise_in_bounds")

        def scal(ref, e):
            return vsplat(ref, e)[0]

        def extract(cl, buf, j0):
            n = scal(cnt_v, cl)
            o = scal(off_v, cl)
            cbase = (cl * _NW + wid) * 128

            def one(e, j):
                u = vsplat(us_v, e)
                lane = u - cbase
                b0 = scal(bs_v, e)
                jmod = j % _RING
                for d16 in range(DIM // _L):
                    v = plsc.load_gather(buf, [d16 * _L + lanes, lane])
                    stg[pl.ds(jmod * DIM + d16 * _L, _L)] = v

                @pl.when(j >= _RING)
                def _():
                    drainO.wait()

                pltpu.async_copy(stg.at[pl.ds(jmod * DIM, DIM)],
                                 out_hbm.at[pl.ds(b0 * DIM, DIM)], semO)
                return j + 1

            return lax.fori_loop(o, o + n, one, j0)

        def step(cl, j):
            @pl.when(cl + (NBUF - 1) < n_cl)
            def _():
                fetch_mod(cl + (NBUF - 1))

            def mk(m):
                def br(j):
                    drains[m].wait()
                    return extract(cl, bufs[m], j)
                return br

            return lax.switch(cl % NBUF, [mk(m) for m in range(NBUF)], j)

        j_end = lax.fori_loop(0, n_cl, step, 0)

        rem = jnp.minimum(j_end, _RING)
        lax.fori_loop(0, rem, lambda i, c: (drainO.wait(), c)[1], 0)


def _dot_kernel(gp_hbm, gq_hbm, out_hbm, p_v, q_v, out_v, sem):
    wid = lax.axis_index("s") * _NC + lax.axis_index("c")
    base = wid * _BPW
    pltpu.sync_copy(gp_hbm.at[pl.ds(base * DIM, _BPW * DIM)], p_v)
    pltpu.sync_copy(gq_hbm.at[pl.ds(base * DIM, _BPW * DIM)], q_v)

    lanes = lax.iota(jnp.int32, _L)
    idx_e = (lanes * 2) % _L
    idx_o = idx_e + 1
    half_m = lanes < (_L // 2)

    def hadd(a, b):
        ae = a.at[idx_e].get(mode="promise_in_bounds")
        ao = a.at[idx_o].get(mode="promise_in_bounds")
        be = b.at[idx_e].get(mode="promise_in_bounds")
        bo = b.at[idx_o].get(mode="promise_in_bounds")
        return jnp.where(half_m, ae + ao, be + bo)

    def body(g, carry):
        accs = []
        for k in range(_L):
            acc = None
            for c in range(DIM // _L):
                o = (g * _L + k) * DIM + c * _L
                pv = p_v[pl.ds(o, _L)]
                qv = q_v[pl.ds(o, _L)]
                acc = pv * qv if acc is None else acc + pv * qv
            accs.append(acc)
        while len(accs) > 1:
            accs = [hadd(accs[i], accs[i + 1]) for i in range(0, len(accs), 2)]
        acc = jnp.minimum(jnp.maximum(accs[0], 0.0), 1.0)
        out_v[pl.ds(g * _L, _L)] = acc
        return carry

    lax.fori_loop(0, _BPW // _L, body, 0)
    pltpu.sync_copy(out_v, out_hbm.at[pl.ds(base, _BPW)])


@jax.jit
def kernel(user_id, item_id, P, Q):
    mesh = plsc.VectorSubcoreMesh(core_axis_name="c", subcore_axis_name="s")
    gather = pl.kernel(
        _gather_kernel,
        out_type=(jax.ShapeDtypeStruct((BATCH * DIM,), jnp.float32),
                  jax.ShapeDtypeStruct((BATCH * DIM,), jnp.float32)),
        mesh=mesh,
        scratch_types=[
            pltpu.VMEM((BATCH,), jnp.int32),
            pltpu.VMEM((BATCH,), jnp.int32),
            pltpu.VMEM((BATCH,), jnp.int32),
            pltpu.VMEM((_CPAD,), jnp.int32),
            pltpu.VMEM((_CPAD,), jnp.int32),
            pltpu.VMEM((_CPAD,), jnp.int32),
            pltpu.VMEM((DIM, 128), jnp.float32),
            pltpu.VMEM((DIM, 128), jnp.float32),
            pltpu.VMEM((DIM, 128), jnp.float32),
            pltpu.VMEM((DIM, 128), jnp.float32),
            pltpu.VMEM((DIM, 128), jnp.float32),
            pltpu.VMEM((DIM, 128), jnp.float32),
            pltpu.VMEM((DIM, 128), jnp.float32),
            pltpu.VMEM((DIM, 128), jnp.float32),
            pltpu.VMEM((_RING * DIM,), jnp.float32),
            pltpu.SemaphoreType.DMA,
            pltpu.SemaphoreType.DMA,
            pltpu.SemaphoreType.DMA,
            pltpu.SemaphoreType.DMA,
            pltpu.SemaphoreType.DMA,
            pltpu.SemaphoreType.DMA,
            pltpu.SemaphoreType.DMA,
            pltpu.SemaphoreType.DMA,
            pltpu.SemaphoreType.DMA,
        ],
        compiler_params=pltpu.CompilerParams(needs_layout_passes=False),
    )
    dot = pl.kernel(
        _dot_kernel,
        out_type=jax.ShapeDtypeStruct((BATCH,), jnp.float32),
        mesh=mesh,
        scratch_types=[
            pltpu.VMEM((_BPW * DIM,), jnp.float32),
            pltpu.VMEM((_BPW * DIM,), jnp.float32),
            pltpu.VMEM((_BPW,), jnp.float32),
            pltpu.SemaphoreType.DMA,
        ],
    )
    gp, gq = gather(user_id.astype(jnp.int32), item_id.astype(jnp.int32),
                    P.T, Q.T)
    score = dot(gp, gq)
    return score[:, None]


# speedup vs baseline: 4.4631x; 1.0848x over previous
import jax
import jax.numpy as jnp
from jax import lax
from jax.experimental import pallas as pl
from jax.experimental.pallas import tpu as pltpu
from jax.experimental.pallas import tpu_sc as plsc

DIM = 64
BATCH = 16384
NU = 1000000

_NC, _NS, _L = 2, 16, 16
_NW = _NC * _NS
_BPW = BATCH // _NW
_NB = (NU + 127) // 128
_MAXCL = (_NB + _NW - 1) // _NW
_CPAD = 256
_RING = 8


def _gather_kernel(uid_hbm, iid_hbm, pt_hbm, qt_hbm, gp_hbm, gq_hbm,
                   idx_v, us_v, bs_v, cnt_v, off_v, cur_v, nz_v,
                   s0, s1, s2, s3, s4, s5, s6, s7, stg,
                   semA, semB, semC, semD, semE, semF, semG, semH, semO):
    wid = lax.axis_index("s") * _NC + lax.axis_index("c")
    lanes = lax.iota(jnp.int32, _L)
    ones = jnp.ones((_L,), jnp.int32)
    zeros = jnp.zeros((_L,), jnp.int32)

    n_cl = (_NB - 1 - wid) // _NW + 1

    bufs = (s0, s1, s2, s3, s4, s5, s6, s7)
    sems = (semA, semB, semC, semD, semE, semF, semG, semH)
    drains = tuple(
        pltpu.make_async_copy(pt_hbm.at[:, pl.ds(0, 128)], b, s)
        for b, s in zip(bufs, sems))
    drainO = pltpu.make_async_copy(
        gp_hbm.at[pl.ds(0, DIM)], stg.at[pl.ds(0, DIM)], semO)

    for src_hbm, tab_hbm, out_hbm in ((uid_hbm, pt_hbm, gp_hbm),
                                      (iid_hbm, qt_hbm, gq_hbm)):
        pltpu.sync_copy(src_hbm, idx_v)

        def zero_cnt(j, carry):
            cnt_v[pl.ds(j * _L, _L)] = zeros
            return carry

        lax.fori_loop(0, _CPAD // _L, zero_cnt, 0)

        def count(i, carry):
            u = idx_v[pl.ds(i * _L, _L)]
            c = u >> 7
            mine = (c & (_NW - 1)) == wid
            cl = c >> 5
            plsc.addupdate_scatter(cnt_v, [cl], ones, mask=mine)
            return carry

        lax.fori_loop(0, BATCH // _L, count, 0)

        def pfx(j, carry):
            chunk = cnt_v[pl.ds(j * _L, _L)]
            cs = plsc.cumsum(chunk)
            off_v[pl.ds(j * _L, _L)] = cs - chunk + carry
            return carry + cs[_L - 1]

        lax.fori_loop(0, _CPAD // _L, pfx, 0)

        def pcopy(j, carry):
            cur_v[pl.ds(j * _L, _L)] = off_v[pl.ds(j * _L, _L)]
            return carry

        lax.fori_loop(0, _CPAD // _L, pcopy, 0)

        def place(i, carry):
            u = idx_v[pl.ds(i * _L, _L)]
            c = u >> 7
            mine = (c & (_NW - 1)) == wid
            cl = c >> 5
            base = plsc.load_gather(cur_v, [cl])
            rank, last = plsc.scan_count(cl, mask=mine)
            pos = base + rank - 1
            plsc.store_scatter(us_v, [pos], u, mask=mine)
            plsc.store_scatter(bs_v, [pos], i * _L + lanes, mask=mine)
            lastm = jnp.logical_and(mine, last)
            plsc.store_scatter(cur_v, [cl], base + rank, mask=lastm)
            return carry

        lax.fori_loop(0, BATCH // _L, place, 0)

        def vsplat(ref, e):
            vec = ref[pl.ds((e // _L) * _L, _L)]
            sel = jnp.full((_L,), 0, jnp.int32) + (e % _L)
            return vec.at[sel].get(mode="promise_in_bounds")

        def scal(ref, e):
            return vsplat(ref, e)[0]

        def compact(j, nzc):
            chunk = cnt_v[pl.ds(j * _L, _L)]
            m = chunk > 0
            plsc.store_compressed(nz_v.at[pl.ds(nzc, _L)], j * _L + lanes,
                                  mask=m)
            npop = plsc.all_reduce_population_count(m)
            return nzc + npop[0]

        n_swp = lax.fori_loop(0, _CPAD // _L, compact, 0)

        NBUF = len(bufs)

        def fetch(i, buf, sem):
            c = scal(nz_v, i) * _NW + wid
            pltpu.async_copy(tab_hbm.at[:, pl.ds(c * 128, 128)], buf, sem)

        def fetch_mod(i):
            for m in range(NBUF):
                @pl.when(i % NBUF == m)
                def _(m=m):
                    fetch(i, bufs[m], sems[m])

        for w in range(NBUF - 1):
            @pl.when(w < n_swp)
            def _(w=w):
                fetch(w, bufs[w], sems[w])

        def extract(i, buf, j0):
            cl = scal(nz_v, i)
            n = scal(cnt_v, cl)
            o = scal(off_v, cl)
            cbase = (cl * _NW + wid) * 128

            def one(e, j):
                u = vsplat(us_v, e)
                lane = u - cbase
                b0 = scal(bs_v, e)
                jmod = j % _RING
                for d16 in range(DIM // _L):
                    v = plsc.load_gather(buf, [d16 * _L + lanes, lane])
                    stg[pl.ds(jmod * DIM + d16 * _L, _L)] = v

                @pl.when(j >= _RING)
                def _():
                    drainO.wait()

                pltpu.async_copy(stg.at[pl.ds(jmod * DIM, DIM)],
                                 out_hbm.at[pl.ds(b0 * DIM, DIM)], semO)
                return j + 1

            return lax.fori_loop(o, o + n, one, j0)

        def step(i, j):
            @pl.when(i + (NBUF - 1) < n_swp)
            def _():
                fetch_mod(i + (NBUF - 1))

            def mk(m):
                def br(j):
                    drains[m].wait()
                    return extract(i, bufs[m], j)
                return br

            return lax.switch(i % NBUF, [mk(m) for m in range(NBUF)], j)

        j_end = lax.fori_loop(0, n_swp, step, 0)

        rem = jnp.minimum(j_end, _RING)
        lax.fori_loop(0, rem, lambda i, c: (drainO.wait(), c)[1], 0)


def _dot_kernel(gp_hbm, gq_hbm, out_hbm, p_v, q_v, out_v, sem):
    wid = lax.axis_index("s") * _NC + lax.axis_index("c")
    base = wid * _BPW
    pltpu.sync_copy(gp_hbm.at[pl.ds(base * DIM, _BPW * DIM)], p_v)
    pltpu.sync_copy(gq_hbm.at[pl.ds(base * DIM, _BPW * DIM)], q_v)

    lanes = lax.iota(jnp.int32, _L)
    idx_e = (lanes * 2) % _L
    idx_o = idx_e + 1
    half_m = lanes < (_L // 2)

    def hadd(a, b):
        ae = a.at[idx_e].get(mode="promise_in_bounds")
        ao = a.at[idx_o].get(mode="promise_in_bounds")
        be = b.at[idx_e].get(mode="promise_in_bounds")
        bo = b.at[idx_o].get(mode="promise_in_bounds")
        return jnp.where(half_m, ae + ao, be + bo)

    def body(g, carry):
        accs = []
        for k in range(_L):
            acc = None
            for c in range(DIM // _L):
                o = (g * _L + k) * DIM + c * _L
                pv = p_v[pl.ds(o, _L)]
                qv = q_v[pl.ds(o, _L)]
                acc = pv * qv if acc is None else acc + pv * qv
            accs.append(acc)
        while len(accs) > 1:
            accs = [hadd(accs[i], accs[i + 1]) for i in range(0, len(accs), 2)]
        acc = jnp.minimum(jnp.maximum(accs[0], 0.0), 1.0)
        out_v[pl.ds(g * _L, _L)] = acc
        return carry

    lax.fori_loop(0, _BPW // _L, body, 0)
    pltpu.sync_copy(out_v, out_hbm.at[pl.ds(base, _BPW)])


@jax.jit
def kernel(user_id, item_id, P, Q):
    mesh = plsc.VectorSubcoreMesh(core_axis_name="c", subcore_axis_name="s")
    gather = pl.kernel(
        _gather_kernel,
        out_type=(jax.ShapeDtypeStruct((BATCH * DIM,), jnp.float32),
                  jax.ShapeDtypeStruct((BATCH * DIM,), jnp.float32)),
        mesh=mesh,
        scratch_types=[
            pltpu.VMEM((BATCH,), jnp.int32),
            pltpu.VMEM((BATCH,), jnp.int32),
            pltpu.VMEM((BATCH,), jnp.int32),
            pltpu.VMEM((_CPAD,), jnp.int32),
            pltpu.VMEM((_CPAD,), jnp.int32),
            pltpu.VMEM((_CPAD,), jnp.int32),
            pltpu.VMEM((_CPAD + _L,), jnp.int32),
            pltpu.VMEM((DIM, 128), jnp.float32),
            pltpu.VMEM((DIM, 128), jnp.float32),
            pltpu.VMEM((DIM, 128), jnp.float32),
            pltpu.VMEM((DIM, 128), jnp.float32),
            pltpu.VMEM((DIM, 128), jnp.float32),
            pltpu.VMEM((DIM, 128), jnp.float32),
            pltpu.VMEM((DIM, 128), jnp.float32),
            pltpu.VMEM((DIM, 128), jnp.float32),
            pltpu.VMEM((_RING * DIM,), jnp.float32),
            pltpu.SemaphoreType.DMA,
            pltpu.SemaphoreType.DMA,
            pltpu.SemaphoreType.DMA,
            pltpu.SemaphoreType.DMA,
            pltpu.SemaphoreType.DMA,
            pltpu.SemaphoreType.DMA,
            pltpu.SemaphoreType.DMA,
            pltpu.SemaphoreType.DMA,
            pltpu.SemaphoreType.DMA,
        ],
        compiler_params=pltpu.CompilerParams(needs_layout_passes=False),
    )
    dot = pl.kernel(
        _dot_kernel,
        out_type=jax.ShapeDtypeStruct((BATCH,), jnp.float32),
        mesh=mesh,
        scratch_types=[
            pltpu.VMEM((_BPW * DIM,), jnp.float32),
            pltpu.VMEM((_BPW * DIM,), jnp.float32),
            pltpu.VMEM((_BPW,), jnp.float32),
            pltpu.SemaphoreType.DMA,
        ],
    )
    gp, gq = gather(user_id.astype(jnp.int32), item_id.astype(jnp.int32),
                    P.T, Q.T)
    score = dot(gp, gq)
    return score[:, None]
